# baseline (speedup 1.0000x reference)
import functools

import jax
import jax.numpy as jnp
from jax import lax
from jax.experimental import pallas as pl
from jax.experimental.pallas import tpu as pltpu
from jax.experimental.pallas import tpu_sc as plsc

D = 768
L = 16
NJ = D // L
EPS = 1e-12
NC = 2
NS = 16
NW = NC * NS
CHUNK = 64


def _lane_sum(x):
    lane = lax.iota(jnp.int32, L)
    dnums = lax.GatherDimensionNumbers(
        offset_dims=(), collapsed_slice_dims=(0,), start_index_map=(0,))
    for k in (8, 4, 2, 1):
        shuf = lax.gather(
            x, (lane ^ k)[:, None], dnums, (1,),
            mode=lax.GatherScatterMode.PROMISE_IN_BOUNDS)
        x = x + shuf
    return x


def _rsqrt_f32(x):
    i = lax.bitcast_convert_type(x, jnp.int32)
    i = jnp.int32(0x5F3759DF) - lax.shift_right_arithmetic(i, 1)
    y = lax.bitcast_convert_type(i, jnp.float32)
    for _ in range(3):
        y = y * (1.5 - 0.5 * x * y * y)
    return y


@functools.partial(jax.jit, static_argnums=(5, 6))
def _run(ids_flat, token_table, pos_table, gamma, beta, total, seq_len):
    tpw = total // NW
    nchunks = tpw // CHUNK
    mesh = plsc.VectorSubcoreMesh(core_axis_name="c", subcore_axis_name="s")

    @functools.partial(
        pl.kernel,
        mesh=mesh,
        out_type=jax.ShapeDtypeStruct((total, D), jnp.float32),
        scratch_types=[
            pltpu.VMEM((tpw,), jnp.int32),
            pltpu.VMEM((CHUNK, D), jnp.float32),
            pltpu.VMEM((CHUNK, D), jnp.float32),
            pltpu.VMEM((2, D), jnp.float32),
            pltpu.SemaphoreType.DMA,
        ],
    )
    def k(ids_hbm, tok_hbm, pos_hbm, gamma_hbm, beta_hbm, out_hbm,
          ids_v, row_v, pos_v, gb_v, sem):
        wid = lax.axis_index("s") * NC + lax.axis_index("c")
        base = wid * tpw
        pos_base = lax.rem(base, seq_len)
        pltpu.sync_copy(gamma_hbm, gb_v.at[0])
        pltpu.sync_copy(beta_hbm, gb_v.at[1])
        pltpu.sync_copy(ids_hbm.at[pl.ds(base, tpw)], ids_v)

        def chunk_body(c, _):
            off = c * CHUNK
            pltpu.sync_copy(pos_hbm.at[pl.ds(pos_base + off, CHUNK)], pos_v)
            pltpu.async_copy(
                tok_hbm.at[ids_v.at[pl.ds(off, CHUNK)]], row_v, sem
            ).wait()

            def row_body(r, _):
                s = jnp.zeros((L,), jnp.float32)
                q = jnp.zeros((L,), jnp.float32)
                for j in range(NJ):
                    sl = pl.ds(j * L, L)
                    e = row_v[r, sl] + pos_v[r, sl]
                    row_v[r, sl] = e
                    s = s + e
                    q = q + e * e
                mean = _lane_sum(s) * (1.0 / D)
                var = _lane_sum(q) * (1.0 / D) - mean * mean
                rinv = _rsqrt_f32(var + EPS)
                for j in range(NJ):
                    sl = pl.ds(j * L, L)
                    row_v[r, sl] = ((row_v[r, sl] - mean) * rinv) * gb_v[0, sl] \
                        + gb_v[1, sl]
                return 0

            lax.fori_loop(0, CHUNK, row_body, 0)
            pltpu.sync_copy(row_v, out_hbm.at[pl.ds(base + off, CHUNK)])
            return 0

        lax.fori_loop(0, nchunks, chunk_body, 0)

    return k(ids_flat, token_table, pos_table, gamma, beta)


def kernel(input_ids, token_table, pos_table, gamma, beta):
    b, s = input_ids.shape
    ids_flat = input_ids.reshape(-1).astype(jnp.int32)
    out = _run(ids_flat, token_table, pos_table, gamma, beta, b * s, s)
    return out.reshape(b, s, D)
